# baseline (speedup 1.0000x reference)
import functools
import math

import jax
import jax.numpy as jnp
from jax import lax
from jax.experimental import pallas as pl
from jax.experimental.pallas import tpu as pltpu
from jax.experimental.pallas import tpu_sc as plsc

XMAX = 100.0
ALPHA = 0.75
NW = 32
CHUNK = 128
GROUP = 16
LN2 = 0.6931471805599453
VBLK = 16384


def _ln(x):
    bits = lax.bitcast_convert_type(x, jnp.int32)
    e = ((bits >> 23) & 0xFF) - 127
    m = lax.bitcast_convert_type(
        (bits & 0x007FFFFF) | jnp.int32(0x3F800000), jnp.float32)
    t = (m - 1.0) / (m + 1.0)
    t2 = t * t
    ln_m = 2.0 * t * (1.0 + t2 * (1.0 / 3.0 + t2 * (
        0.2 + t2 * (1.0 / 7.0 + t2 * (1.0 / 9.0)))))
    return e.astype(jnp.float32) * LN2 + ln_m


def _fuse_body(wt_ref, wtp_ref, out_ref):
    out_ref[...] = jnp.concatenate(
        [wt_ref[...].T, wtp_ref[...].T], axis=1)


def _sc_body(ppw, E, i_hbm, j_hbm, x_hbm, ww_hbm, b_hbm, bp_hbm,
             out_hbm, idx_i, idx_j, wi2, wj2, bi_v, bj_v, x_v, acc_v,
             sem_r, sem_b):
    nch = ppw // CHUNK
    c = lax.axis_index("c")
    s = lax.axis_index("s")
    wid = s * 2 + c

    pltpu.sync_copy(i_hbm.at[wid], idx_i)
    pltpu.sync_copy(j_hbm.at[wid], idx_j)
    pltpu.sync_copy(x_hbm.at[wid], x_v)

    bias_copies = []
    for ch in range(nch):
        bias_copies.append((
            pltpu.async_copy(
                b_hbm.at[idx_i.at[pl.ds(ch * CHUNK, CHUNK)]], bi_v.at[ch],
                sem_b[ch]),
            pltpu.async_copy(
                bp_hbm.at[idx_j.at[pl.ds(ch * CHUNK, CHUNK)]], bj_v.at[ch],
                sem_b[ch]),
        ))

    def fire_rows(ch):
        p = ch % 2
        return (
            pltpu.async_copy(
                ww_hbm.at[idx_i.at[pl.ds(ch * CHUNK, CHUNK)]], wi2.at[p],
                sem_r[p]),
            pltpu.async_copy(
                ww_hbm.at[idx_j.at[pl.ds(ch * CHUNK, CHUNK)]], wj2.at[p],
                sem_r[p]),
        )

    inflight = fire_rows(0)

    lanes = lax.iota(jnp.int32, GROUP)
    rot_idx = [(lanes + sh) % GROUP for sh in (8, 4, 2, 1)]
    ln_xmax = math.log(XMAX)

    def hsum(p):
        for idx in rot_idx:
            p = p + p.at[idx].get(mode="promise_in_bounds")
        return p

    def make_group_step(ch):
        p = ch % 2

        def group_step(gg, acc):
            row0 = gg * GROUP
            base = ch * CHUNK + row0
            dots = jnp.zeros((GROUP,), jnp.float32)
            for rr in range(GROUP):
                r = row0 + rr
                pr = wi2[p, r, pl.ds(0, 16)] * wj2[p, r, pl.ds(E, 16)]
                for k in range(1, E // 16):
                    pr = pr + (wi2[p, r, pl.ds(16 * k, 16)] *
                               wj2[p, r, pl.ds(E + 16 * k, 16)])
                dots = jnp.where(lanes == rr, hsum(pr), dots)
            bi = bi_v[ch, pl.ds(row0, 16)]
            bj = bj_v[ch, pl.ds(row0, 16)]
            xg = x_v[pl.ds(base, 16)]
            lnx = _ln(xg)
            cfg = jnp.minimum(jnp.exp(ALPHA * (lnx - ln_xmax)), 1.0)
            err = dots + bi + bj - lnx
            return acc + cfg * err * err
        return group_step

    acc = jnp.zeros((GROUP,), jnp.float32)
    for ch in range(nch):
        for cp in inflight:
            cp.wait()
        if ch + 1 < nch:
            inflight = fire_rows(ch + 1)
        for cp in bias_copies[ch]:
            cp.wait()
        acc = lax.fori_loop(0, CHUNK // GROUP, make_group_step(ch), acc)

    acc_v[...] = acc
    pltpu.sync_copy(acc_v, out_hbm.at[wid])


def kernel(i, j, xij, W, W_, b, b_):
    B = i.shape[0]
    V, E = W.shape
    ppw = B // NW
    nch = ppw // CHUNK
    grid = (V + VBLK - 1) // VBLK

    ww = pl.pallas_call(
        _fuse_body,
        grid=(grid,),
        in_specs=[
            pl.BlockSpec((E, VBLK), lambda p: (0, p)),
            pl.BlockSpec((E, VBLK), lambda p: (0, p)),
        ],
        out_specs=pl.BlockSpec((VBLK, 2 * E), lambda p: (p, 0)),
        out_shape=jax.ShapeDtypeStruct((V, 2 * E), jnp.float32),
    )(W.T, W_.T)

    i2 = jnp.asarray(i, jnp.int32).reshape(NW, ppw)
    j2 = jnp.asarray(j, jnp.int32).reshape(NW, ppw)
    x2 = xij.reshape(NW, ppw)

    mesh = plsc.VectorSubcoreMesh(core_axis_name="c", subcore_axis_name="s")
    sc = functools.partial(
        pl.kernel,
        mesh=mesh,
        compiler_params=pltpu.CompilerParams(use_tc_tiling_on_sc=True),
        out_type=jax.ShapeDtypeStruct((NW, GROUP), jnp.float32),
        scratch_types=[
            pltpu.VMEM((ppw,), jnp.int32),
            pltpu.VMEM((ppw,), jnp.int32),
            pltpu.VMEM((2, CHUNK, 2 * E), jnp.float32),
            pltpu.VMEM((2, CHUNK, 2 * E), jnp.float32),
            pltpu.VMEM((nch, CHUNK), jnp.float32),
            pltpu.VMEM((nch, CHUNK), jnp.float32),
            pltpu.VMEM((ppw,), jnp.float32),
            pltpu.VMEM((GROUP,), jnp.float32),
            [pltpu.SemaphoreType.DMA] * 2,
            [pltpu.SemaphoreType.DMA] * nch,
        ],
    )(functools.partial(_sc_body, ppw, E))

    partials = sc(i2, j2, x2, ww, b, b_)
    return jnp.sum(partials) / B


# speedup vs baseline: 1.2620x; 1.2620x over previous
import functools
import math

import jax
import jax.numpy as jnp
from jax import lax
from jax.experimental import pallas as pl
from jax.experimental.pallas import tpu as pltpu
from jax.experimental.pallas import tpu_sc as plsc

XMAX = 100.0
ALPHA = 0.75
NW = 32
CHUNK = 128
GROUP = 16
LN2 = 0.6931471805599453
VBLK = 8192
HMASK = -0x10000
RND = 0x8000


def _ln(x):
    bits = lax.bitcast_convert_type(x, jnp.int32)
    e = ((bits >> 23) & 0xFF) - 127
    m = lax.bitcast_convert_type(
        (bits & 0x007FFFFF) | jnp.int32(0x3F800000), jnp.float32)
    t = (m - 1.0) / (m + 1.0)
    t2 = t * t
    ln_m = 2.0 * t * (1.0 + t2 * (1.0 / 3.0 + t2 * (
        0.2 + t2 * (1.0 / 7.0 + t2 * (1.0 / 9.0)))))
    return e.astype(jnp.float32) * LN2 + ln_m


def _fuse_body(wt_ref, wtp_ref, out_ref):
    lo = lax.shift_right_logical(
        lax.bitcast_convert_type(wt_ref[...], jnp.int32) + RND, 16)
    hi = (lax.bitcast_convert_type(wtp_ref[...], jnp.int32) + RND) & HMASK
    packed = hi | lo
    h = packed.shape[1] // 2
    out_ref[...] = jnp.concatenate(
        [packed[:, :h].T, packed[:, h:].T], axis=1)


def _sc_body(ppw, E, i_hbm, j_hbm, x_hbm, ww_hbm, b_hbm, bp_hbm,
             out_hbm, idx_i, idx_j, idx_ih, idx_jh, wi2, wj2, bi_v, bj_v,
             x_v, acc_v, sem_r, sem_b):
    nch = ppw // CHUNK
    c = lax.axis_index("c")
    s = lax.axis_index("s")
    wid = s * 2 + c

    pltpu.sync_copy(i_hbm.at[wid], idx_i)
    pltpu.sync_copy(j_hbm.at[wid], idx_j)
    pltpu.sync_copy(x_hbm.at[wid], x_v)

    sh = VBLK.bit_length() - 1
    hm = VBLK // 2 - 1
    for o in range(0, ppw, GROUP):
        vi = idx_i[pl.ds(o, GROUP)]
        vj = idx_j[pl.ds(o, GROUP)]
        idx_ih[pl.ds(o, GROUP)] = (
            lax.shift_left(lax.shift_right_logical(vi, sh), sh - 1)
            | (vi & hm))
        idx_jh[pl.ds(o, GROUP)] = (
            lax.shift_left(lax.shift_right_logical(vj, sh), sh - 1)
            | (vj & hm))

    bias_copies = []
    for ch in range(nch):
        bias_copies.append((
            pltpu.async_copy(
                b_hbm.at[idx_i.at[pl.ds(ch * CHUNK, CHUNK)]], bi_v.at[ch],
                sem_b[ch]),
            pltpu.async_copy(
                bp_hbm.at[idx_j.at[pl.ds(ch * CHUNK, CHUNK)]], bj_v.at[ch],
                sem_b[ch]),
        ))

    def fire_rows(ch):
        p = ch % 2
        return (
            pltpu.async_copy(
                ww_hbm.at[idx_ih.at[pl.ds(ch * CHUNK, CHUNK)]], wi2.at[p],
                sem_r[p]),
            pltpu.async_copy(
                ww_hbm.at[idx_jh.at[pl.ds(ch * CHUNK, CHUNK)]], wj2.at[p],
                sem_r[p]),
        )

    inflight = fire_rows(0)

    lanes = lax.iota(jnp.int32, GROUP)
    rot_idx = [(lanes + sh) % GROUP for sh in (8, 4, 2, 1)]
    ln_xmax = math.log(XMAX)

    def hsum(p):
        for idx in rot_idx:
            p = p + p.at[idx].get(mode="promise_in_bounds")
        return p

    def make_group_step(ch):
        p = ch % 2

        def group_step(gg, acc):
            row0 = gg * GROUP
            base = ch * CHUNK + row0
            oiv = (lax.shift_right_logical(
                idx_i[pl.ds(base, GROUP)], sh - 1) & 1) * E
            ojv = (lax.shift_right_logical(
                idx_j[pl.ds(base, GROUP)], sh - 1) & 1) * E
            dots = jnp.zeros((GROUP,), jnp.float32)
            for rr in range(GROUP):
                r = row0 + rr
                oi = oiv[rr]
                oj = ojv[rr]
                pr = None
                for k in range(E // 16):
                    a = wi2[p, r, pl.ds(oi + 16 * k, 16)]
                    bq = wj2[p, r, pl.ds(oj + 16 * k, 16)]
                    wa = lax.bitcast_convert_type(
                        lax.shift_left(a, 16), jnp.float32)
                    wb = lax.bitcast_convert_type(bq & HMASK, jnp.float32)
                    pr = wa * wb if pr is None else pr + wa * wb
                dots = jnp.where(lanes == rr, hsum(pr), dots)
            bi = bi_v[ch, pl.ds(row0, 16)]
            bj = bj_v[ch, pl.ds(row0, 16)]
            xg = x_v[pl.ds(base, 16)]
            lnx = _ln(xg)
            cfg = jnp.minimum(jnp.exp(ALPHA * (lnx - ln_xmax)), 1.0)
            err = dots + bi + bj - lnx
            return acc + cfg * err * err
        return group_step

    acc = jnp.zeros((GROUP,), jnp.float32)
    for ch in range(nch):
        for cp in inflight:
            cp.wait()
        if ch + 1 < nch:
            inflight = fire_rows(ch + 1)
        for cp in bias_copies[ch]:
            cp.wait()
        acc = lax.fori_loop(0, CHUNK // GROUP, make_group_step(ch), acc)

    acc_v[...] = acc
    pltpu.sync_copy(acc_v, out_hbm.at[wid])


def kernel(i, j, xij, W, W_, b, b_):
    B = i.shape[0]
    V, E = W.shape
    ppw = B // NW
    nch = ppw // CHUNK
    grid = (V + VBLK - 1) // VBLK

    ww = pl.pallas_call(
        _fuse_body,
        grid=(grid,),
        in_specs=[
            pl.BlockSpec((E, VBLK), lambda p: (0, p)),
            pl.BlockSpec((E, VBLK), lambda p: (0, p)),
        ],
        out_specs=pl.BlockSpec((VBLK // 2, 2 * E), lambda p: (p, 0)),
        out_shape=jax.ShapeDtypeStruct((grid * (VBLK // 2), 2 * E),
                                       jnp.int32),
    )(W.T, W_.T)

    i2 = jnp.asarray(i, jnp.int32).reshape(NW, ppw)
    j2 = jnp.asarray(j, jnp.int32).reshape(NW, ppw)
    x2 = xij.reshape(NW, ppw)

    mesh = plsc.VectorSubcoreMesh(core_axis_name="c", subcore_axis_name="s")
    sc = functools.partial(
        pl.kernel,
        mesh=mesh,
        compiler_params=pltpu.CompilerParams(use_tc_tiling_on_sc=True),
        out_type=jax.ShapeDtypeStruct((NW, GROUP), jnp.float32),
        scratch_types=[
            pltpu.VMEM((ppw,), jnp.int32),
            pltpu.VMEM((ppw,), jnp.int32),
            pltpu.VMEM((ppw,), jnp.int32),
            pltpu.VMEM((ppw,), jnp.int32),
            pltpu.VMEM((2, CHUNK, 2 * E), jnp.int32),
            pltpu.VMEM((2, CHUNK, 2 * E), jnp.int32),
            pltpu.VMEM((nch, CHUNK), jnp.float32),
            pltpu.VMEM((nch, CHUNK), jnp.float32),
            pltpu.VMEM((ppw,), jnp.float32),
            pltpu.VMEM((GROUP,), jnp.float32),
            [pltpu.SemaphoreType.DMA] * 2,
            [pltpu.SemaphoreType.DMA] * nch,
        ],
    )(functools.partial(_sc_body, ppw, E))

    partials = sc(i2, j2, x2, ww, b, b_)
    return jnp.sum(partials) / B


# speedup vs baseline: 1.2775x; 1.0123x over previous
import functools
import math

import jax
import jax.numpy as jnp
from jax import lax
from jax.experimental import pallas as pl
from jax.experimental.pallas import tpu as pltpu
from jax.experimental.pallas import tpu_sc as plsc

XMAX = 100.0
ALPHA = 0.75
NW = 32
CHUNK = 128
GROUP = 16
LN2 = 0.6931471805599453
VBLK = 16384
HMASK = -0x10000
RND = 0x8000


def _ln(x):
    bits = lax.bitcast_convert_type(x, jnp.int32)
    e = ((bits >> 23) & 0xFF) - 127
    m = lax.bitcast_convert_type(
        (bits & 0x007FFFFF) | jnp.int32(0x3F800000), jnp.float32)
    t = (m - 1.0) / (m + 1.0)
    t2 = t * t
    ln_m = 2.0 * t * (1.0 + t2 * (1.0 / 3.0 + t2 * (
        0.2 + t2 * (1.0 / 7.0 + t2 * (1.0 / 9.0)))))
    return e.astype(jnp.float32) * LN2 + ln_m


def _fuse_body(wt_ref, wtp_ref, out_ref):
    lo = lax.shift_right_logical(
        lax.bitcast_convert_type(wt_ref[...], jnp.int32) + RND, 16)
    hi = (lax.bitcast_convert_type(wtp_ref[...], jnp.int32) + RND) & HMASK
    packed = hi | lo
    h = packed.shape[1] // 2
    out_ref[...] = jnp.concatenate(
        [packed[:, :h].T, packed[:, h:].T], axis=1)


def _sc_body(ppw, E, i_hbm, j_hbm, x_hbm, ww_hbm, b_hbm, bp_hbm,
             out_hbm, idx_i, idx_j, idx_ih, idx_jh, wi2, wj2, bi_v, bj_v,
             x_v, acc_v, sem_r, sem_b):
    nch = ppw // CHUNK
    c = lax.axis_index("c")
    s = lax.axis_index("s")
    wid = s * 2 + c

    pltpu.sync_copy(i_hbm.at[wid], idx_i)
    pltpu.sync_copy(j_hbm.at[wid], idx_j)
    pltpu.sync_copy(x_hbm.at[wid], x_v)

    sh = VBLK.bit_length() - 1
    hm = VBLK // 2 - 1
    for o in range(0, ppw, GROUP):
        vi = idx_i[pl.ds(o, GROUP)]
        vj = idx_j[pl.ds(o, GROUP)]
        idx_ih[pl.ds(o, GROUP)] = (
            lax.shift_left(lax.shift_right_logical(vi, sh), sh - 1)
            | (vi & hm))
        idx_jh[pl.ds(o, GROUP)] = (
            lax.shift_left(lax.shift_right_logical(vj, sh), sh - 1)
            | (vj & hm))

    bias_copies = []
    for ch in range(nch):
        bias_copies.append((
            pltpu.async_copy(
                b_hbm.at[idx_i.at[pl.ds(ch * CHUNK, CHUNK)]], bi_v.at[ch],
                sem_b[ch]),
            pltpu.async_copy(
                bp_hbm.at[idx_j.at[pl.ds(ch * CHUNK, CHUNK)]], bj_v.at[ch],
                sem_b[ch]),
        ))

    def fire_rows(ch):
        p = ch % 2
        return (
            pltpu.async_copy(
                ww_hbm.at[idx_ih.at[pl.ds(ch * CHUNK, CHUNK)]], wi2.at[p],
                sem_r[p]),
            pltpu.async_copy(
                ww_hbm.at[idx_jh.at[pl.ds(ch * CHUNK, CHUNK)]], wj2.at[p],
                sem_r[p]),
        )

    inflight = fire_rows(0)

    lanes = lax.iota(jnp.int32, GROUP)
    rot_idx = [(lanes + sh) % GROUP for sh in (8, 4, 2, 1)]
    ln_xmax = math.log(XMAX)

    def hsum(p):
        for idx in rot_idx:
            p = p + p.at[idx].get(mode="promise_in_bounds")
        return p

    def make_group_step(ch):
        p = ch % 2

        def group_step(gg, acc):
            row0 = gg * GROUP
            base = ch * CHUNK + row0
            oiv = (lax.shift_right_logical(
                idx_i[pl.ds(base, GROUP)], sh - 1) & 1) * E
            ojv = (lax.shift_right_logical(
                idx_j[pl.ds(base, GROUP)], sh - 1) & 1) * E
            dots = jnp.zeros((GROUP,), jnp.float32)
            for rr in range(GROUP):
                r = row0 + rr
                oi = oiv[rr]
                oj = ojv[rr]
                pr = None
                for k in range(E // 16):
                    a = wi2[p, r, pl.ds(oi + 16 * k, 16)]
                    bq = wj2[p, r, pl.ds(oj + 16 * k, 16)]
                    wa = lax.bitcast_convert_type(
                        lax.shift_left(a, 16), jnp.float32)
                    wb = lax.bitcast_convert_type(bq & HMASK, jnp.float32)
                    pr = wa * wb if pr is None else pr + wa * wb
                dots = jnp.where(lanes == rr, hsum(pr), dots)
            bi = bi_v[ch, pl.ds(row0, 16)]
            bj = bj_v[ch, pl.ds(row0, 16)]
            xg = x_v[pl.ds(base, 16)]
            lnx = _ln(xg)
            cfg = jnp.minimum(jnp.exp(ALPHA * (lnx - ln_xmax)), 1.0)
            err = dots + bi + bj - lnx
            return acc + cfg * err * err
        return group_step

    acc = jnp.zeros((GROUP,), jnp.float32)
    for ch in range(nch):
        for cp in inflight:
            cp.wait()
        if ch + 1 < nch:
            inflight = fire_rows(ch + 1)
        for cp in bias_copies[ch]:
            cp.wait()
        acc = lax.fori_loop(0, CHUNK // GROUP, make_group_step(ch), acc)

    acc_v[...] = acc
    pltpu.sync_copy(acc_v, out_hbm.at[wid])


def kernel(i, j, xij, W, W_, b, b_):
    B = i.shape[0]
    V, E = W.shape
    ppw = B // NW
    nch = ppw // CHUNK
    grid = (V + VBLK - 1) // VBLK

    ww = pl.pallas_call(
        _fuse_body,
        grid=(grid,),
        in_specs=[
            pl.BlockSpec((E, VBLK), lambda p: (0, p)),
            pl.BlockSpec((E, VBLK), lambda p: (0, p)),
        ],
        out_specs=pl.BlockSpec((VBLK // 2, 2 * E), lambda p: (p, 0)),
        out_shape=jax.ShapeDtypeStruct((grid * (VBLK // 2), 2 * E),
                                       jnp.int32),
    )(W.T, W_.T)

    i2 = jnp.asarray(i, jnp.int32).reshape(NW, ppw)
    j2 = jnp.asarray(j, jnp.int32).reshape(NW, ppw)
    x2 = xij.reshape(NW, ppw)

    mesh = plsc.VectorSubcoreMesh(core_axis_name="c", subcore_axis_name="s")
    sc = functools.partial(
        pl.kernel,
        mesh=mesh,
        compiler_params=pltpu.CompilerParams(use_tc_tiling_on_sc=True),
        out_type=jax.ShapeDtypeStruct((NW, GROUP), jnp.float32),
        scratch_types=[
            pltpu.VMEM((ppw,), jnp.int32),
            pltpu.VMEM((ppw,), jnp.int32),
            pltpu.VMEM((ppw,), jnp.int32),
            pltpu.VMEM((ppw,), jnp.int32),
            pltpu.VMEM((2, CHUNK, 2 * E), jnp.int32),
            pltpu.VMEM((2, CHUNK, 2 * E), jnp.int32),
            pltpu.VMEM((nch, CHUNK), jnp.float32),
            pltpu.VMEM((nch, CHUNK), jnp.float32),
            pltpu.VMEM((ppw,), jnp.float32),
            pltpu.VMEM((GROUP,), jnp.float32),
            [pltpu.SemaphoreType.DMA] * 2,
            [pltpu.SemaphoreType.DMA] * nch,
        ],
    )(functools.partial(_sc_body, ppw, E))

    partials = sc(i2, j2, x2, ww, b, b_)
    return jnp.sum(partials) / B
